# baseline (speedup 1.0000x reference)
import functools

import jax
import jax.numpy as jnp
from jax import lax
from jax.experimental import pallas as pl
from jax.experimental.pallas import tpu as pltpu
from jax.experimental.pallas import tpu_sc as plsc

_N = 10000
_E = 320000
_D = 128
_NC = 2
_NS = 16
_NW = _NC * _NS
_L = 16

_CHUNK = 128
_ITERS = 80
_EPAD = _NW * _ITERS * _CHUNK
_STRIPE = 624


def _mm_body(x_ref, w_ref, o_ref):
    o_ref[...] = jnp.dot(x_ref[...], w_ref[...],
                         preferred_element_type=jnp.float32)


def _matmul(x, w):
    return pl.pallas_call(
        _mm_body,
        grid=(5,),
        in_specs=[
            pl.BlockSpec((2000, _D), lambda i: (i, 0)),
            pl.BlockSpec((_D, _D), lambda i: (0, 0)),
        ],
        out_specs=pl.BlockSpec((2000, _D), lambda i: (i, 0)),
        out_shape=jax.ShapeDtypeStruct((_N, _D), jnp.float32),
    )(x, w)


_mesh = plsc.VectorSubcoreMesh(core_axis_name="c", subcore_axis_name="s")


@functools.partial(
    pl.kernel,
    out_type=jax.ShapeDtypeStruct((_NC, _N, _D), jnp.float32),
    mesh=_mesh,
    scratch_types=[
        pltpu.VMEM((_CHUNK,), jnp.int32),
        pltpu.VMEM((_CHUNK,), jnp.int32),
        pltpu.VMEM((_CHUNK,), jnp.int32),
        pltpu.VMEM((_CHUNK,), jnp.int32),
        pltpu.VMEM((_CHUNK,), jnp.float32),
        pltpu.VMEM((_CHUNK,), jnp.float32),
        pltpu.VMEM((_CHUNK, _D), jnp.float32),
        pltpu.VMEM((_CHUNK, _D), jnp.float32),
        pltpu.VMEM_SHARED((_N, _D), jnp.float32),
        pltpu.SemaphoreType.DMA,
        pltpu.SemaphoreType.DMA,
        pltpu.SemaphoreType.DMA,
        pltpu.SemaphoreType.DMA,
        pltpu.SemaphoreType.DMA,
        pltpu.SemaphoreType.DMA,
    ],
)
def _spmm(src_hbm, dst_hbm, ew_hbm, sup_hbm, out_hbm,
          src0, src1, dst0, dst1, w0, w1, rows0, rows1, acc,
          ls0, ls1, ldw0, ldw1, gat0, gat1):
    c = lax.axis_index("c")
    s = lax.axis_index("s")
    wid = s * _NC + c
    srcs = (src0, src1)
    dsts = (dst0, dst1)
    ws = (w0, w1)
    rows = (rows0, rows1)
    lds = (ls0, ls1)
    lddw = (ldw0, ldw1)
    gat = (gat0, gat1)
    ebase = wid * (_ITERS * _CHUNK)

    def _start_src(b, it):
        base = ebase + it * _CHUNK
        pltpu.async_copy(src_hbm.at[pl.ds(base, _CHUNK)], srcs[b], lds[b])

    def _wait_src(b):
        pltpu.make_async_copy(src_hbm.at[pl.ds(0, _CHUNK)], srcs[b],
                              lds[b]).wait()

    def _start_dw(b, it):
        base = ebase + it * _CHUNK
        pltpu.async_copy(dst_hbm.at[pl.ds(base, _CHUNK)], dsts[b], lddw[b])
        pltpu.async_copy(ew_hbm.at[pl.ds(base, _CHUNK)], ws[b], lddw[b])

    def _wait_dw(b):
        pltpu.make_async_copy(dst_hbm.at[pl.ds(0, _CHUNK)], dsts[b],
                              lddw[b]).wait()
        pltpu.make_async_copy(ew_hbm.at[pl.ds(0, _CHUNK)], ws[b],
                              lddw[b]).wait()

    def _start_gather(b):
        pltpu.async_copy(sup_hbm.at[srcs[b]], rows[b], gat[b])

    def _wait_gather(b):
        pltpu.make_async_copy(sup_hbm.at[srcs[b]], rows[b], gat[b]).wait()

    def _scale(b):
        rb = rows[b]
        wb = ws[b]

        def _scale16(g, carry):
            wvec = wb[pl.ds(g * _L, _L)]
            for l in range(_L):
                wl = wvec.at[jnp.full((_L,), l, jnp.int32)].get(
                    mode="promise_in_bounds")
                r = g * _L + l
                for j in range(_D // _L):
                    sl = pl.ds(j * _L, _L)
                    rb[r, sl] = rb[r, sl] * wl
            return carry
        lax.fori_loop(0, _CHUNK // _L, _scale16, 0)

    _start_src(0, 0)
    _start_dw(0, 0)
    _start_src(1, 1)
    _start_dw(1, 1)

    def _zero_row(i, carry):
        for j in range(_D // _L):
            rows0[i, pl.ds(j * _L, _L)] = jnp.zeros((_L,), jnp.float32)
        return carry
    lax.fori_loop(0, _CHUNK, _zero_row, 0)

    stripe = s * _STRIPE
    for off in (0, 128, 256, 384, 496):
        pltpu.sync_copy(rows0, acc.at[pl.ds(stripe + off, _CHUNK)])

    @pl.when(s == _NS - 1)
    def _zero_tail():
        pltpu.sync_copy(rows0, acc.at[pl.ds(_N - _CHUNK, _CHUNK)])
    plsc.subcore_barrier()

    _wait_src(0)
    _start_gather(0)

    def _body(t, carry):
        it0 = t * 2
        it1 = it0 + 1

        _wait_src(1)
        _start_gather(1)
        _wait_gather(0)

        @pl.when(it0 + 2 < _ITERS)
        def _():
            _start_src(0, it0 + 2)
        _wait_dw(0)
        _scale(0)
        pltpu.sync_copy(rows[0], acc.at[dsts[0]], add=True)

        @pl.when(it0 + 2 < _ITERS)
        def _():
            _start_dw(0, it0 + 2)

        @pl.when(it1 + 1 < _ITERS)
        def _():
            _wait_src(0)
            _start_gather(0)
        _wait_gather(1)

        @pl.when(it1 + 2 < _ITERS)
        def _():
            _start_src(1, it1 + 2)
        _wait_dw(1)
        _scale(1)
        pltpu.sync_copy(rows[1], acc.at[dsts[1]], add=True)

        @pl.when(it1 + 2 < _ITERS)
        def _():
            _start_dw(1, it1 + 2)
        return carry
    lax.fori_loop(0, _ITERS // 2, _body, 0)

    plsc.subcore_barrier()
    for off in (0, 128, 256, 384, 496):
        pltpu.sync_copy(acc.at[pl.ds(stripe + off, _CHUNK)],
                        out_hbm.at[c, pl.ds(stripe + off, _CHUNK)])

    @pl.when(s == _NS - 1)
    def _write_tail():
        pltpu.sync_copy(acc.at[pl.ds(_N - _CHUNK, _CHUNK)],
                        out_hbm.at[c, pl.ds(_N - _CHUNK, _CHUNK)])


def _comb_body(p_ref, b_ref, o_ref):
    o_ref[...] = p_ref[0] + p_ref[1] + b_ref[...]


def _combine(partials, bias2d):
    return pl.pallas_call(
        _comb_body,
        grid=(5,),
        in_specs=[
            pl.BlockSpec((_NC, 2000, _D), lambda i: (0, i, 0)),
            pl.BlockSpec((1, _D), lambda i: (0, 0)),
        ],
        out_specs=pl.BlockSpec((2000, _D), lambda i: (i, 0)),
        out_shape=jax.ShapeDtypeStruct((_N, _D), jnp.float32),
    )(partials, bias2d)


def kernel(x, edge_index, edge_weight, weight, bias):
    support = _matmul(x, weight)
    pad = _EPAD - _E
    ei = jnp.pad(edge_index, ((0, 0), (0, pad)))
    ew = jnp.pad(edge_weight, (0, pad))
    partials = _spmm(ei[0], ei[1], ew, support)
    return _combine(partials, bias.reshape(1, _D))
